# Initial kernel scaffold; baseline (speedup 1.0000x reference)
import jax
import jax.numpy as jnp
from jax.experimental import pallas as pl


def kernel(pos_hits_xyz, feat):
    raise NotImplementedError("write your pallas kernel here")


# speedup vs baseline: 15.8385x; 15.8385x over previous
import functools

import jax
import jax.numpy as jnp
from jax import lax
from jax.experimental import pallas as pl

_N = 16384
_SIDE = 128
_NUP = _N // 4
_M = 7
_BIG_I = 2**30


def _r2(op, x):
    return op(op(x, axis=1, keepdims=True), axis=0, keepdims=True)


def _fps_kernel(px_ref, py_ref, pz_ref, mask_ref):
    px = px_ref[...]
    py = py_ref[...]
    pz = pz_ref[...]
    ir = lax.broadcasted_iota(jnp.int32, (_SIDE, _SIDE), 0)
    ic = lax.broadcasted_iota(jnp.int32, (_SIDE, _SIDE), 1)
    flat = ir * _SIDE + ic

    def dist_to(eq):
        xb = _r2(jnp.sum, jnp.where(eq, px, 0.0))
        yb = _r2(jnp.sum, jnp.where(eq, py, 0.0))
        zb = _r2(jnp.sum, jnp.where(eq, pz, 0.0))
        dx = px - xb
        dy = py - yb
        dz = pz - zb
        return (dx * dx + dy * dy) + dz * dz

    eq0 = flat == 0
    mask = jnp.where(eq0, 1.0, 0.0)
    d_min = dist_to(eq0)

    def body(_, carry):
        d_min, mask = carry
        m = _r2(jnp.max, d_min)
        best = _r2(jnp.min, jnp.where(d_min == m, flat, _BIG_I))
        eq = flat == best
        mask = jnp.where(eq, 1.0, mask)
        d_min = jnp.minimum(d_min, dist_to(eq))
        return d_min, mask

    _, mask = lax.fori_loop(1, _NUP, body, (d_min, mask))
    mask_ref[...] = mask


def _fps_mask(px2, py2, pz2):
    return pl.pallas_call(
        _fps_kernel,
        out_shape=jax.ShapeDtypeStruct((_SIDE, _SIDE), jnp.float32),
    )(px2, py2, pz2)


def _knn_kernel(u_ref, c_ref, dist_ref, idx_ref, *, rows, cols, exclude_self):
    ux = u_ref[:, 0:1]
    uy = u_ref[:, 1:2]
    uz = u_ref[:, 2:3]
    cx = c_ref[0:1, :]
    cy = c_ref[1:2, :]
    cz = c_ref[2:3, :]
    aa = (ux * ux + uy * uy) + uz * uz
    bb = (cx * cx + cy * cy) + cz * cz
    def _b(v):
        return v.astype(jnp.bfloat16).astype(jnp.float32)
    ab = (_b(ux) * _b(cx) + _b(uy) * _b(cy)) + _b(uz) * _b(cz)
    d2 = (aa - 2.0 * ab) + bb
    colio = lax.broadcasted_iota(jnp.int32, (rows, cols), 1)
    if exclude_self:
        rowio = lax.broadcasted_iota(jnp.int32, (rows, cols), 0)
        d2 = jnp.where(colio == rowio + pl.program_id(0) * rows, jnp.inf, d2)
    for k in range(_M):
        m = jnp.min(d2, axis=1, keepdims=True)
        im = jnp.min(jnp.where(d2 == m, colio, _BIG_I), axis=1, keepdims=True)
        dist_ref[:, k:k + 1] = m
        idx_ref[:, k:k + 1] = im
        if k < _M - 1:
            d2 = jnp.where(colio == im, jnp.inf, d2)
    dist_ref[:, _M:_M + 1] = jnp.zeros((rows, 1), jnp.float32)
    idx_ref[:, _M:_M + 1] = jnp.zeros((rows, 1), jnp.int32)


def _knn(u8, c8, rows, exclude_self):
    nu, cols = u8.shape[0], c8.shape[1]
    dist, idx = pl.pallas_call(
        functools.partial(_knn_kernel, rows=rows, cols=cols,
                          exclude_self=exclude_self),
        grid=(nu // rows,),
        in_specs=[
            pl.BlockSpec((rows, 8), lambda i: (i, 0)),
            pl.BlockSpec((8, cols), lambda i: (0, 0)),
        ],
        out_specs=[
            pl.BlockSpec((rows, 8), lambda i: (i, 0)),
            pl.BlockSpec((rows, 8), lambda i: (i, 0)),
        ],
        out_shape=[
            jax.ShapeDtypeStruct((nu, 8), jnp.float32),
            jax.ShapeDtypeStruct((nu, 8), jnp.int32),
        ],
    )(u8, c8)
    return dist[:, :_M], idx[:, :_M]


def kernel(pos_hits_xyz, feat):
    pos = pos_hits_xyz
    px2 = pos[:, 0].reshape(_SIDE, _SIDE)
    py2 = pos[:, 1].reshape(_SIDE, _SIDE)
    pz2 = pos[:, 2].reshape(_SIDE, _SIDE)

    center = _fps_mask(px2, py2, pz2).reshape(-1)
    maskb = center.astype(bool)

    order = jnp.argsort(~maskb, stable=True)
    nodes_up = order[:_NUP]
    nodes_down = order[_NUP:]
    up = jnp.take(pos, nodes_up, axis=0)
    down = jnp.take(pos, nodes_down, axis=0)

    u8 = jnp.pad(up, ((0, 0), (0, 5)))
    c8_down = jnp.pad(down.T, ((0, 5), (0, 0)))
    c8_up = jnp.pad(up.T, ((0, 5), (0, 0)))

    ndist, nidx = _knn(u8, c8_down, rows=256, exclude_self=False)
    _, uidx = _knn(u8, c8_up, rows=256, exclude_self=True)

    j = jnp.take(nodes_down, nidx.reshape(-1)).astype(jnp.int64)
    i = jnp.repeat(nodes_up, _M).astype(jnp.int64)
    edge_src_up = uidx.reshape(-1).astype(jnp.int64)
    edge_dst_up = jnp.repeat(jnp.arange(_NUP), _M).astype(jnp.int64)

    return (j, i, pos, center, feat, edge_src_up, edge_dst_up, ndist)


# speedup vs baseline: 16.3289x; 1.0310x over previous
import functools

import jax
import jax.numpy as jnp
from jax import lax
from jax.experimental import pallas as pl

_N = 16384
_SIDE = 128
_NUP = _N // 4
_M = 7
_BIG_I = 2**30


def _r2(op, x):
    return op(op(x, axis=0, keepdims=True), axis=1, keepdims=True)


def _fps_kernel(px_ref, py_ref, pz_ref, mask_ref):
    px = px_ref[...]
    py = py_ref[...]
    pz = pz_ref[...]
    ir = lax.broadcasted_iota(jnp.int32, (_SIDE, _SIDE), 0)
    ic = lax.broadcasted_iota(jnp.int32, (_SIDE, _SIDE), 1)
    flat = ir * _SIDE + ic

    def dist_to(eq):
        xb = _r2(jnp.sum, jnp.where(eq, px, 0.0))
        yb = _r2(jnp.sum, jnp.where(eq, py, 0.0))
        zb = _r2(jnp.sum, jnp.where(eq, pz, 0.0))
        dx = px - xb
        dy = py - yb
        dz = pz - zb
        return (dx * dx + dy * dy) + dz * dz

    eq0 = flat == 0
    mask = jnp.where(eq0, 1.0, 0.0)
    d_min = dist_to(eq0)

    def body(_, carry):
        d_min, mask = carry
        m = _r2(jnp.max, d_min)
        best = _r2(jnp.min, jnp.where(d_min == m, flat, _BIG_I))
        eq = flat == best
        mask = jnp.where(eq, 1.0, mask)
        d_min = jnp.minimum(d_min, dist_to(eq))
        return d_min, mask

    _, mask = lax.fori_loop(1, _NUP, body, (d_min, mask))
    mask_ref[...] = mask


def _fps_mask(px2, py2, pz2):
    return pl.pallas_call(
        _fps_kernel,
        out_shape=jax.ShapeDtypeStruct((_SIDE, _SIDE), jnp.float32),
    )(px2, py2, pz2)


def _knn_kernel(u_ref, c_ref, dist_ref, idx_ref, *, rows, cols, exclude_self):
    ux = u_ref[:, 0:1]
    uy = u_ref[:, 1:2]
    uz = u_ref[:, 2:3]
    cx = c_ref[0:1, :]
    cy = c_ref[1:2, :]
    cz = c_ref[2:3, :]
    aa = (ux * ux + uy * uy) + uz * uz
    bb = (cx * cx + cy * cy) + cz * cz
    def _b(v):
        return v.astype(jnp.bfloat16).astype(jnp.float32)
    ab = (_b(ux) * _b(cx) + _b(uy) * _b(cy)) + _b(uz) * _b(cz)
    d2 = (aa - 2.0 * ab) + bb
    colio = lax.broadcasted_iota(jnp.int32, (rows, cols), 1)
    if exclude_self:
        rowio = lax.broadcasted_iota(jnp.int32, (rows, cols), 0)
        d2 = jnp.where(colio == rowio + pl.program_id(0) * rows, jnp.inf, d2)
    for k in range(_M):
        m = jnp.min(d2, axis=1, keepdims=True)
        im = jnp.min(jnp.where(d2 == m, colio, _BIG_I), axis=1, keepdims=True)
        dist_ref[:, k:k + 1] = m
        idx_ref[:, k:k + 1] = im
        if k < _M - 1:
            d2 = jnp.where(colio == im, jnp.inf, d2)
    dist_ref[:, _M:_M + 1] = jnp.zeros((rows, 1), jnp.float32)
    idx_ref[:, _M:_M + 1] = jnp.zeros((rows, 1), jnp.int32)


def _knn(u8, c8, rows, exclude_self):
    nu, cols = u8.shape[0], c8.shape[1]
    dist, idx = pl.pallas_call(
        functools.partial(_knn_kernel, rows=rows, cols=cols,
                          exclude_self=exclude_self),
        grid=(nu // rows,),
        in_specs=[
            pl.BlockSpec((rows, 8), lambda i: (i, 0)),
            pl.BlockSpec((8, cols), lambda i: (0, 0)),
        ],
        out_specs=[
            pl.BlockSpec((rows, 8), lambda i: (i, 0)),
            pl.BlockSpec((rows, 8), lambda i: (i, 0)),
        ],
        out_shape=[
            jax.ShapeDtypeStruct((nu, 8), jnp.float32),
            jax.ShapeDtypeStruct((nu, 8), jnp.int32),
        ],
    )(u8, c8)
    return dist[:, :_M], idx[:, :_M]


def kernel(pos_hits_xyz, feat):
    pos = pos_hits_xyz
    px2 = pos[:, 0].reshape(_SIDE, _SIDE)
    py2 = pos[:, 1].reshape(_SIDE, _SIDE)
    pz2 = pos[:, 2].reshape(_SIDE, _SIDE)

    center = _fps_mask(px2, py2, pz2).reshape(-1)
    maski = center.astype(jnp.int32)

    cs = jnp.cumsum(maski)
    ranks = jnp.where(maski == 1, cs - 1,
                      _NUP + jnp.arange(_N, dtype=jnp.int32) - cs)
    order = jnp.zeros((_N,), jnp.int32).at[ranks].set(
        jnp.arange(_N, dtype=jnp.int32))
    nodes_up = order[:_NUP]
    nodes_down = order[_NUP:]
    up = jnp.take(pos, nodes_up, axis=0)
    down = jnp.take(pos, nodes_down, axis=0)

    u8 = jnp.pad(up, ((0, 0), (0, 5)))
    c8_down = jnp.pad(down.T, ((0, 5), (0, 0)))
    c8_up = jnp.pad(up.T, ((0, 5), (0, 0)))

    ndist, nidx = _knn(u8, c8_down, rows=256, exclude_self=False)
    _, uidx = _knn(u8, c8_up, rows=256, exclude_self=True)

    j = jnp.take(nodes_down, nidx.reshape(-1)).astype(jnp.int64)
    i = jnp.repeat(nodes_up, _M).astype(jnp.int64)
    edge_src_up = uidx.reshape(-1).astype(jnp.int64)
    edge_dst_up = jnp.repeat(jnp.arange(_NUP), _M).astype(jnp.int64)

    return (j, i, pos, center, feat, edge_src_up, edge_dst_up, ndist)


# speedup vs baseline: 16.3733x; 1.0027x over previous
import functools

import jax
import jax.numpy as jnp
from jax import lax
from jax.experimental import pallas as pl

_N = 16384
_SIDE = 128
_NUP = _N // 4
_M = 7
_BIG_I = 2**30


def _r2(op, x):
    return op(op(x, axis=0, keepdims=True), axis=1, keepdims=True)


def _fps_kernel(px_ref, py_ref, pz_ref, idx_ref):
    px = px_ref[...]
    py = py_ref[...]
    pz = pz_ref[...]
    ir = lax.broadcasted_iota(jnp.int32, (_SIDE, _SIDE), 0)
    ic = lax.broadcasted_iota(jnp.int32, (_SIDE, _SIDE), 1)
    flat = ir * _SIDE + ic

    def argmax2d(d):
        dd, ii = d, flat
        rows = _SIDE
        while rows > 1:
            h = rows // 2
            t = (dd[:h] > dd[h:]) | ((dd[:h] == dd[h:]) & (ii[:h] < ii[h:]))
            dd = jnp.where(t, dd[:h], dd[h:])
            ii = jnp.where(t, ii[:h], ii[h:])
            rows = h
        m = jnp.max(dd, axis=1, keepdims=True)
        ist = jnp.min(jnp.where(dd == m, ii, _BIG_I), axis=1, keepdims=True)
        return ist

    def dist_to(eq):
        xb = _r2(jnp.sum, jnp.where(eq, px, 0.0))
        yb = _r2(jnp.sum, jnp.where(eq, py, 0.0))
        zb = _r2(jnp.sum, jnp.where(eq, pz, 0.0))
        dx = px - xb
        dy = py - yb
        dz = pz - zb
        return (dx * dx + dy * dy) + dz * dz

    idx_ref[0:1, 0:1] = jnp.zeros((1, 1), jnp.int32)
    d_min = dist_to(flat == 0)

    def body(i, d_min):
        ist = argmax2d(d_min)
        idx_ref[pl.ds(i, 1), 0:1] = ist
        return jnp.minimum(d_min, dist_to(flat == ist))

    lax.fori_loop(1, _NUP, body, d_min)


def _fps_indices(px2, py2, pz2):
    return pl.pallas_call(
        _fps_kernel,
        out_shape=jax.ShapeDtypeStruct((_NUP, 1), jnp.int32),
    )(px2, py2, pz2)


def _knn_kernel(u_ref, c_ref, dist_ref, idx_ref, *, rows, cols, exclude_self):
    ux = u_ref[:, 0:1]
    uy = u_ref[:, 1:2]
    uz = u_ref[:, 2:3]
    cx = c_ref[0:1, :]
    cy = c_ref[1:2, :]
    cz = c_ref[2:3, :]
    aa = (ux * ux + uy * uy) + uz * uz
    bb = (cx * cx + cy * cy) + cz * cz
    def _b(v):
        return v.astype(jnp.bfloat16).astype(jnp.float32)
    ab = (_b(ux) * _b(cx) + _b(uy) * _b(cy)) + _b(uz) * _b(cz)
    d2 = (aa - 2.0 * ab) + bb
    colio = lax.broadcasted_iota(jnp.int32, (rows, cols), 1)
    if exclude_self:
        rowio = lax.broadcasted_iota(jnp.int32, (rows, cols), 0)
        d2 = jnp.where(colio == rowio + pl.program_id(0) * rows, jnp.inf, d2)
    for k in range(_M):
        m = jnp.min(d2, axis=1, keepdims=True)
        im = jnp.min(jnp.where(d2 == m, colio, _BIG_I), axis=1, keepdims=True)
        dist_ref[:, k:k + 1] = m
        idx_ref[:, k:k + 1] = im
        if k < _M - 1:
            d2 = jnp.where(colio == im, jnp.inf, d2)
    dist_ref[:, _M:_M + 1] = jnp.zeros((rows, 1), jnp.float32)
    idx_ref[:, _M:_M + 1] = jnp.zeros((rows, 1), jnp.int32)


def _knn(u8, c8, rows, exclude_self):
    nu, cols = u8.shape[0], c8.shape[1]
    dist, idx = pl.pallas_call(
        functools.partial(_knn_kernel, rows=rows, cols=cols,
                          exclude_self=exclude_self),
        grid=(nu // rows,),
        in_specs=[
            pl.BlockSpec((rows, 8), lambda i: (i, 0)),
            pl.BlockSpec((8, cols), lambda i: (0, 0)),
        ],
        out_specs=[
            pl.BlockSpec((rows, 8), lambda i: (i, 0)),
            pl.BlockSpec((rows, 8), lambda i: (i, 0)),
        ],
        out_shape=[
            jax.ShapeDtypeStruct((nu, 8), jnp.float32),
            jax.ShapeDtypeStruct((nu, 8), jnp.int32),
        ],
    )(u8, c8)
    return dist[:, :_M], idx[:, :_M]


def kernel(pos_hits_xyz, feat):
    pos = pos_hits_xyz
    px2 = pos[:, 0].reshape(_SIDE, _SIDE)
    py2 = pos[:, 1].reshape(_SIDE, _SIDE)
    pz2 = pos[:, 2].reshape(_SIDE, _SIDE)

    centroids = _fps_indices(px2, py2, pz2).reshape(-1)
    center = jnp.zeros((_N,), jnp.float32).at[centroids].set(1.0)
    maski = center.astype(jnp.int32)

    cs = jnp.cumsum(maski)
    ranks = jnp.where(maski == 1, cs - 1,
                      _NUP + jnp.arange(_N, dtype=jnp.int32) - cs)
    order = jnp.zeros((_N,), jnp.int32).at[ranks].set(
        jnp.arange(_N, dtype=jnp.int32))
    nodes_up = order[:_NUP]
    nodes_down = order[_NUP:]
    up = jnp.take(pos, nodes_up, axis=0)
    down = jnp.take(pos, nodes_down, axis=0)

    u8 = jnp.pad(up, ((0, 0), (0, 5)))
    c8_down = jnp.pad(down.T, ((0, 5), (0, 0)))
    c8_up = jnp.pad(up.T, ((0, 5), (0, 0)))

    ndist, nidx = _knn(u8, c8_down, rows=256, exclude_self=False)
    _, uidx = _knn(u8, c8_up, rows=256, exclude_self=True)

    j = jnp.take(nodes_down, nidx.reshape(-1)).astype(jnp.int64)
    i = jnp.repeat(nodes_up, _M).astype(jnp.int64)
    edge_src_up = uidx.reshape(-1).astype(jnp.int64)
    edge_dst_up = jnp.repeat(jnp.arange(_NUP), _M).astype(jnp.int64)

    return (j, i, pos, center, feat, edge_src_up, edge_dst_up, ndist)
